# baseline (speedup 1.0000x reference)
import functools

import jax
import jax.numpy as jnp
from jax import lax
from jax.experimental import pallas as pl
from jax.experimental.pallas import tpu as pltpu
from jax.experimental.pallas import tpu_sc as plsc

_BM = 512

_SC_CORES = 2
_SC_SUBCORES = 16
_SC_WORKERS = _SC_CORES * _SC_SUBCORES
_LANES = 16


def _ffn_tile_kernel(blocks_ref, experts_ref, lo_ref, hi_ref,
                     x_ref, wg_ref, wu_ref, wd_ref, s_ref, out_ref):
    t = pl.program_id(0)
    lo = lo_ref[t]
    hi = hi_ref[t]
    base = blocks_ref[t] * _BM
    x = x_ref[...]
    g = jnp.dot(x, wg_ref[0], preferred_element_type=jnp.float32)
    u = jnp.dot(x, wu_ref[0], preferred_element_type=jnp.float32)
    h = (g * jax.nn.sigmoid(g)) * u
    o = jnp.dot(h.astype(jnp.bfloat16), wd_ref[0],
                preferred_element_type=jnp.float32)
    rows = base + jax.lax.broadcasted_iota(jnp.int32, (_BM, 1), 0)
    val = jnp.where((rows >= lo) & (rows < hi), o * s_ref[...], 0.0)

    @pl.when(lo == base)
    def _init():
        out_ref[...] = val

    @pl.when(lo != base)
    def _accum():
        out_ref[...] = out_ref[...] + val


def _grouped_ffn(sorted_x, sorted_scores, step_block, step_expert, step_lo,
                 step_hi, W_gate, W_up, W_down):
    n, d_model = sorted_x.shape
    d_ff = W_gate.shape[2]
    num_steps = step_block.shape[0]

    grid_spec = pltpu.PrefetchScalarGridSpec(
        num_scalar_prefetch=4,
        grid=(num_steps,),
        in_specs=[
            pl.BlockSpec((_BM, d_model), lambda t, bl, ex, lo, hi: (bl[t], 0)),
            pl.BlockSpec((1, d_model, d_ff), lambda t, bl, ex, lo, hi: (ex[t], 0, 0)),
            pl.BlockSpec((1, d_model, d_ff), lambda t, bl, ex, lo, hi: (ex[t], 0, 0)),
            pl.BlockSpec((1, d_ff, d_model), lambda t, bl, ex, lo, hi: (ex[t], 0, 0)),
            pl.BlockSpec((_BM, 1), lambda t, bl, ex, lo, hi: (bl[t], 0)),
        ],
        out_specs=pl.BlockSpec((_BM, d_model), lambda t, bl, ex, lo, hi: (bl[t], 0)),
    )
    return pl.pallas_call(
        _ffn_tile_kernel,
        grid_spec=grid_spec,
        out_shape=jax.ShapeDtypeStruct((n, d_model), jnp.float32),
    )(step_block, step_expert, step_lo, step_hi, sorted_x, W_gate, W_up,
      W_down, sorted_scores)


def _sc_combine(cat, pos0, pos1):
    b = pos0.shape[0]
    d = cat.shape[1]
    per_w = b // _SC_WORKERS
    ch = min(64, per_w)
    n_ch = per_w // ch
    groups = d // _LANES
    mesh = plsc.VectorSubcoreMesh(core_axis_name="c", subcore_axis_name="s")

    @functools.partial(
        pl.kernel, mesh=mesh,
        out_type=jax.ShapeDtypeStruct((b, d), jnp.float32),
        scratch_types=[
            pltpu.VMEM((ch,), jnp.int32),
            pltpu.VMEM((ch,), jnp.int32),
            pltpu.VMEM((ch, d), jnp.float32),
            pltpu.VMEM((ch, d), jnp.float32),
            pltpu.SemaphoreType.DMA,
        ],
    )
    def k(cat_hbm, pos0_hbm, pos1_hbm, y_hbm, idx0_v, idx1_v, buf0, buf1, sem):
        wid = lax.axis_index("s") * _SC_CORES + lax.axis_index("c")
        wbase = wid * per_w

        def chunk_body(c, carry):
            base = wbase + c * ch
            pltpu.sync_copy(pos0_hbm.at[pl.ds(base, ch)], idx0_v)
            pltpu.sync_copy(pos1_hbm.at[pl.ds(base, ch)], idx1_v)
            pltpu.async_copy(cat_hbm.at[idx0_v], buf0, sem).wait()
            pltpu.async_copy(cat_hbm.at[idx1_v], buf1, sem).wait()

            def tok_body(r, carry2):
                for j in range(groups):
                    a = buf0[r, pl.ds(j * _LANES, _LANES)]
                    bb = buf1[r, pl.ds(j * _LANES, _LANES)]
                    buf0[r, pl.ds(j * _LANES, _LANES)] = a + bb
                return carry2

            lax.fori_loop(0, ch, tok_body, 0)
            pltpu.sync_copy(buf0, y_hbm.at[pl.ds(base, ch)])
            return carry

        lax.fori_loop(0, n_ch, chunk_body, 0)

    return k(cat, pos0, pos1)


def kernel(x, topK_indices, topK_scores, W_gate, W_up, W_down):
    batch, k = topK_indices.shape
    num_experts = W_gate.shape[0]
    n = batch * k

    flat_idx = topK_indices.reshape(-1)
    order = jnp.argsort(flat_idx)
    sorted_idx = flat_idx[order]
    sorted_batch = order // k

    counts = jnp.bincount(flat_idx, length=num_experts)
    off = jnp.concatenate([jnp.zeros((1,), jnp.int32),
                           jnp.cumsum(counts).astype(jnp.int32)])

    nb = n // _BM
    e_first = sorted_idx[::_BM]
    e_last = sorted_idx[_BM - 1::_BM]
    nb_tiles = (e_last - e_first + 1).astype(jnp.int32)
    start_t = jnp.concatenate([jnp.zeros((1,), jnp.int32),
                               jnp.cumsum(nb_tiles).astype(jnp.int32)[:-1]])
    total = start_t[-1] + nb_tiles[-1]

    num_steps = nb + num_experts
    ts = jnp.arange(num_steps, dtype=jnp.int32)
    blk = jnp.searchsorted(start_t, ts, side='right').astype(jnp.int32) - 1
    blk = jnp.clip(blk, 0, nb - 1)
    expert = e_first[blk] + (ts - start_t[blk])
    valid = ts < total
    expert = jnp.clip(jnp.where(valid, expert, sorted_idx[-1]), 0, num_experts - 1)
    base = blk * _BM
    lo = jnp.maximum(base, off[expert])
    hi = jnp.minimum(base + _BM, off[expert + 1])
    hi = jnp.maximum(hi, lo)
    lo = jnp.where(valid, lo, base + _BM)
    hi = jnp.where(valid, hi, base + _BM)

    sorted_x = x.astype(jnp.bfloat16)[sorted_batch]
    sorted_scores = topK_scores.reshape(-1)[order].reshape(n, 1)
    cat = _grouped_ffn(sorted_x, sorted_scores, blk, expert, lo, hi,
                       W_gate.astype(jnp.bfloat16), W_up.astype(jnp.bfloat16),
                       W_down.astype(jnp.bfloat16))

    inv = jnp.zeros((n,), jnp.int32).at[order].set(jnp.arange(n, dtype=jnp.int32))
    pos = inv.reshape(batch, k)
    return _sc_combine(cat, pos[:, 0], pos[:, 1])


# speedup vs baseline: 1.8965x; 1.8965x over previous
import functools

import jax
import jax.numpy as jnp
from jax import lax
from jax.experimental import pallas as pl
from jax.experimental.pallas import tpu as pltpu
from jax.experimental.pallas import tpu_sc as plsc

_BM = 256

_SC_CORES = 2
_SC_SUBCORES = 16
_SC_WORKERS = _SC_CORES * _SC_SUBCORES
_LANES = 16


def _ffn_tile_kernel(blocks_ref, experts_ref, lo_ref, hi_ref,
                     x_ref, wg_ref, wu_ref, wd_ref, s_ref, out_ref):
    t = pl.program_id(0)
    lo = lo_ref[t]
    hi = hi_ref[t]
    base = blocks_ref[t] * _BM
    x = x_ref[...]
    g = jnp.dot(x, wg_ref[0], preferred_element_type=jnp.float32)
    u = jnp.dot(x, wu_ref[0], preferred_element_type=jnp.float32)
    h = (g * jax.nn.sigmoid(g)) * u
    o = jnp.dot(h, wd_ref[0], preferred_element_type=jnp.float32)
    rows = base + jax.lax.broadcasted_iota(jnp.int32, (_BM, 1), 0)
    val = jnp.where((rows >= lo) & (rows < hi), o * s_ref[...], 0.0)

    @pl.when(lo == base)
    def _init():
        out_ref[...] = val

    @pl.when(lo != base)
    def _accum():
        out_ref[...] = out_ref[...] + val


def _grouped_ffn(sorted_x, sorted_scores, step_block, step_expert, step_lo,
                 step_hi, W_gate, W_up, W_down):
    n, d_model = sorted_x.shape
    d_ff = W_gate.shape[2]
    num_steps = step_block.shape[0]

    grid_spec = pltpu.PrefetchScalarGridSpec(
        num_scalar_prefetch=4,
        grid=(num_steps,),
        in_specs=[
            pl.BlockSpec((_BM, d_model), lambda t, bl, ex, lo, hi: (bl[t], 0)),
            pl.BlockSpec((1, d_model, d_ff), lambda t, bl, ex, lo, hi: (ex[t], 0, 0)),
            pl.BlockSpec((1, d_model, d_ff), lambda t, bl, ex, lo, hi: (ex[t], 0, 0)),
            pl.BlockSpec((1, d_ff, d_model), lambda t, bl, ex, lo, hi: (ex[t], 0, 0)),
            pl.BlockSpec((_BM, 1), lambda t, bl, ex, lo, hi: (bl[t], 0)),
        ],
        out_specs=pl.BlockSpec((_BM, d_model), lambda t, bl, ex, lo, hi: (bl[t], 0)),
    )
    return pl.pallas_call(
        _ffn_tile_kernel,
        grid_spec=grid_spec,
        out_shape=jax.ShapeDtypeStruct((n, d_model), jnp.float32),
    )(step_block, step_expert, step_lo, step_hi, sorted_x, W_gate, W_up,
      W_down, sorted_scores)


def _sc_combine(cat, pos0, pos1):
    b = pos0.shape[0]
    d = cat.shape[1]
    per_w = b // _SC_WORKERS
    ch = min(64, per_w)
    n_ch = per_w // ch
    groups = d // _LANES
    mesh = plsc.VectorSubcoreMesh(core_axis_name="c", subcore_axis_name="s")

    @functools.partial(
        pl.kernel, mesh=mesh,
        out_type=jax.ShapeDtypeStruct((b, d), jnp.float32),
        scratch_types=[
            pltpu.VMEM((ch,), jnp.int32),
            pltpu.VMEM((ch,), jnp.int32),
            pltpu.VMEM((ch, d), jnp.float32),
            pltpu.VMEM((ch, d), jnp.float32),
            pltpu.SemaphoreType.DMA,
        ],
    )
    def k(cat_hbm, pos0_hbm, pos1_hbm, y_hbm, idx0_v, idx1_v, buf0, buf1, sem):
        wid = lax.axis_index("s") * _SC_CORES + lax.axis_index("c")
        wbase = wid * per_w

        def chunk_body(c, carry):
            base = wbase + c * ch
            pltpu.sync_copy(pos0_hbm.at[pl.ds(base, ch)], idx0_v)
            pltpu.sync_copy(pos1_hbm.at[pl.ds(base, ch)], idx1_v)
            pltpu.async_copy(cat_hbm.at[idx0_v], buf0, sem).wait()
            pltpu.async_copy(cat_hbm.at[idx1_v], buf1, sem).wait()

            def tok_body(r, carry2):
                for j in range(groups):
                    a = buf0[r, pl.ds(j * _LANES, _LANES)]
                    bb = buf1[r, pl.ds(j * _LANES, _LANES)]
                    buf0[r, pl.ds(j * _LANES, _LANES)] = a + bb
                return carry2

            lax.fori_loop(0, ch, tok_body, 0)
            pltpu.sync_copy(buf0, y_hbm.at[pl.ds(base, ch)])
            return carry

        lax.fori_loop(0, n_ch, chunk_body, 0)

    return k(cat, pos0, pos1)


def kernel(x, topK_indices, topK_scores, W_gate, W_up, W_down):
    batch, k = topK_indices.shape
    num_experts = W_gate.shape[0]
    n = batch * k

    flat_idx = topK_indices.reshape(-1)
    order = jnp.argsort(flat_idx)
    sorted_idx = flat_idx[order]
    sorted_batch = order // k

    counts = jnp.bincount(flat_idx, length=num_experts)
    off = jnp.concatenate([jnp.zeros((1,), jnp.int32),
                           jnp.cumsum(counts).astype(jnp.int32)])

    nb = n // _BM
    e_first = sorted_idx[::_BM]
    e_last = sorted_idx[_BM - 1::_BM]
    nb_tiles = (e_last - e_first + 1).astype(jnp.int32)
    start_t = jnp.concatenate([jnp.zeros((1,), jnp.int32),
                               jnp.cumsum(nb_tiles).astype(jnp.int32)[:-1]])
    total = start_t[-1] + nb_tiles[-1]

    num_steps = nb + num_experts
    ts = jnp.arange(num_steps, dtype=jnp.int32)
    blk = jnp.searchsorted(start_t, ts, side='right').astype(jnp.int32) - 1
    blk = jnp.clip(blk, 0, nb - 1)
    expert = e_first[blk] + (ts - start_t[blk])
    valid = ts < total
    expert = jnp.clip(jnp.where(valid, expert, sorted_idx[-1]), 0, num_experts - 1)
    base = blk * _BM
    lo = jnp.maximum(base, off[expert])
    hi = jnp.minimum(base + _BM, off[expert + 1])
    hi = jnp.maximum(hi, lo)
    lo = jnp.where(valid, lo, base + _BM)
    hi = jnp.where(valid, hi, base + _BM)

    sorted_x = x[sorted_batch]
    sorted_scores = topK_scores.reshape(-1)[order].reshape(n, 1)
    cat = _grouped_ffn(sorted_x, sorted_scores, blk, expert, lo, hi,
                       W_gate, W_up, W_down)

    inv = jnp.zeros((n,), jnp.int32).at[order].set(jnp.arange(n, dtype=jnp.int32))
    pos = inv.reshape(batch, k)
    return _sc_combine(cat, pos[:, 0], pos[:, 1])


# speedup vs baseline: 2.0062x; 1.0578x over previous
import functools

import jax
import jax.numpy as jnp
from jax import lax
from jax.experimental import pallas as pl
from jax.experimental.pallas import tpu as pltpu
from jax.experimental.pallas import tpu_sc as plsc

_BM = 512

_SC_CORES = 2
_SC_SUBCORES = 16
_SC_WORKERS = _SC_CORES * _SC_SUBCORES
_LANES = 16


def _ffn_tile_kernel(blocks_ref, experts_ref, lo_ref, hi_ref,
                     x_ref, wg_ref, wu_ref, wd_ref, s_ref, out_ref):
    t = pl.program_id(0)
    lo = lo_ref[t]
    hi = hi_ref[t]
    base = blocks_ref[t] * _BM
    x = x_ref[...]
    g = jnp.dot(x, wg_ref[0], preferred_element_type=jnp.float32)
    u = jnp.dot(x, wu_ref[0], preferred_element_type=jnp.float32)
    h = (g * jax.nn.sigmoid(g)) * u
    o = jnp.dot(h, wd_ref[0], preferred_element_type=jnp.float32)
    rows = base + jax.lax.broadcasted_iota(jnp.int32, (_BM, 1), 0)
    val = jnp.where((rows >= lo) & (rows < hi), o * s_ref[...], 0.0)

    @pl.when(lo == base)
    def _init():
        out_ref[...] = val

    @pl.when(lo != base)
    def _accum():
        out_ref[...] = out_ref[...] + val


def _grouped_ffn(sorted_x, sorted_scores, step_block, step_expert, step_lo,
                 step_hi, W_gate, W_up, W_down):
    n, d_model = sorted_x.shape
    d_ff = W_gate.shape[2]
    num_steps = step_block.shape[0]

    grid_spec = pltpu.PrefetchScalarGridSpec(
        num_scalar_prefetch=4,
        grid=(num_steps,),
        in_specs=[
            pl.BlockSpec((_BM, d_model), lambda t, bl, ex, lo, hi: (bl[t], 0)),
            pl.BlockSpec((1, d_model, d_ff), lambda t, bl, ex, lo, hi: (ex[t], 0, 0)),
            pl.BlockSpec((1, d_model, d_ff), lambda t, bl, ex, lo, hi: (ex[t], 0, 0)),
            pl.BlockSpec((1, d_ff, d_model), lambda t, bl, ex, lo, hi: (ex[t], 0, 0)),
            pl.BlockSpec((_BM, 1), lambda t, bl, ex, lo, hi: (bl[t], 0)),
        ],
        out_specs=pl.BlockSpec((_BM, d_model), lambda t, bl, ex, lo, hi: (bl[t], 0)),
    )
    return pl.pallas_call(
        _ffn_tile_kernel,
        grid_spec=grid_spec,
        out_shape=jax.ShapeDtypeStruct((n, d_model), jnp.float32),
    )(step_block, step_expert, step_lo, step_hi, sorted_x, W_gate, W_up,
      W_down, sorted_scores)


def _sc_combine(cat, pos0, pos1):
    b = pos0.shape[0]
    d = cat.shape[1]
    per_w = b // _SC_WORKERS
    ch = min(64, per_w)
    n_ch = per_w // ch
    groups = d // _LANES
    mesh = plsc.VectorSubcoreMesh(core_axis_name="c", subcore_axis_name="s")

    @functools.partial(
        pl.kernel, mesh=mesh,
        out_type=jax.ShapeDtypeStruct((b, d), jnp.float32),
        scratch_types=[
            pltpu.VMEM((ch,), jnp.int32),
            pltpu.VMEM((ch,), jnp.int32),
            pltpu.VMEM((ch, d), jnp.float32),
            pltpu.VMEM((ch, d), jnp.float32),
            pltpu.SemaphoreType.DMA,
        ],
    )
    def k(cat_hbm, pos0_hbm, pos1_hbm, y_hbm, idx0_v, idx1_v, buf0, buf1, sem):
        wid = lax.axis_index("s") * _SC_CORES + lax.axis_index("c")
        wbase = wid * per_w

        def chunk_body(c, carry):
            base = wbase + c * ch
            pltpu.sync_copy(pos0_hbm.at[pl.ds(base, ch)], idx0_v)
            pltpu.sync_copy(pos1_hbm.at[pl.ds(base, ch)], idx1_v)
            cp0 = pltpu.async_copy(cat_hbm.at[idx0_v], buf0, sem)
            cp1 = pltpu.async_copy(cat_hbm.at[idx1_v], buf1, sem)
            cp0.wait()
            cp1.wait()

            def tok_body(r, carry2):
                for j in range(groups):
                    a = buf0[r, pl.ds(j * _LANES, _LANES)]
                    bb = buf1[r, pl.ds(j * _LANES, _LANES)]
                    buf0[r, pl.ds(j * _LANES, _LANES)] = a + bb
                return carry2

            lax.fori_loop(0, ch, tok_body, 0)
            pltpu.sync_copy(buf0, y_hbm.at[pl.ds(base, ch)])
            return carry

        lax.fori_loop(0, n_ch, chunk_body, 0)

    return k(cat, pos0, pos1)


def kernel(x, topK_indices, topK_scores, W_gate, W_up, W_down):
    batch, k = topK_indices.shape
    num_experts = W_gate.shape[0]
    n = batch * k

    flat_idx = topK_indices.reshape(-1)
    order = jnp.argsort(flat_idx)
    sorted_idx = flat_idx[order]
    sorted_batch = order // k

    counts = jnp.bincount(flat_idx, length=num_experts)
    off = jnp.concatenate([jnp.zeros((1,), jnp.int32),
                           jnp.cumsum(counts).astype(jnp.int32)])

    nb = n // _BM
    e_first = sorted_idx[::_BM]
    e_last = sorted_idx[_BM - 1::_BM]
    nb_tiles = (e_last - e_first + 1).astype(jnp.int32)
    start_t = jnp.concatenate([jnp.zeros((1,), jnp.int32),
                               jnp.cumsum(nb_tiles).astype(jnp.int32)[:-1]])
    total = start_t[-1] + nb_tiles[-1]

    num_steps = nb + num_experts
    ts = jnp.arange(num_steps, dtype=jnp.int32)
    blk = jnp.searchsorted(start_t, ts, side='right').astype(jnp.int32) - 1
    blk = jnp.clip(blk, 0, nb - 1)
    expert = e_first[blk] + (ts - start_t[blk])
    valid = ts < total
    expert = jnp.clip(jnp.where(valid, expert, sorted_idx[-1]), 0, num_experts - 1)
    base = blk * _BM
    lo = jnp.maximum(base, off[expert])
    hi = jnp.minimum(base + _BM, off[expert + 1])
    hi = jnp.maximum(hi, lo)
    lo = jnp.where(valid, lo, base + _BM)
    hi = jnp.where(valid, hi, base + _BM)

    sorted_x = x[sorted_batch]
    sorted_scores = topK_scores.reshape(-1)[order].reshape(n, 1)
    cat = _grouped_ffn(sorted_x, sorted_scores, blk, expert, lo, hi,
                       W_gate, W_up, W_down)

    inv = jnp.zeros((n,), jnp.int32).at[order].set(jnp.arange(n, dtype=jnp.int32))
    pos = inv.reshape(batch, k)
    return _sc_combine(cat, pos[:, 0], pos[:, 1])


# speedup vs baseline: 2.0113x; 1.0025x over previous
import functools

import jax
import jax.numpy as jnp
from jax import lax
from jax.experimental import pallas as pl
from jax.experimental.pallas import tpu as pltpu
from jax.experimental.pallas import tpu_sc as plsc

_BM = 1024

_SC_CORES = 2
_SC_SUBCORES = 16
_SC_WORKERS = _SC_CORES * _SC_SUBCORES
_LANES = 16


def _ffn_tile_kernel(blocks_ref, experts_ref, lo_ref, hi_ref,
                     x_ref, wg_ref, wu_ref, wd_ref, s_ref, out_ref):
    t = pl.program_id(0)
    lo = lo_ref[t]
    hi = hi_ref[t]
    base = blocks_ref[t] * _BM
    x = x_ref[...]
    g = jnp.dot(x, wg_ref[0], preferred_element_type=jnp.float32)
    u = jnp.dot(x, wu_ref[0], preferred_element_type=jnp.float32)
    h = (g * jax.nn.sigmoid(g)) * u
    o = jnp.dot(h, wd_ref[0], preferred_element_type=jnp.float32)
    rows = base + jax.lax.broadcasted_iota(jnp.int32, (_BM, 1), 0)
    val = jnp.where((rows >= lo) & (rows < hi), o * s_ref[...], 0.0)

    @pl.when(lo == base)
    def _init():
        out_ref[...] = val

    @pl.when(lo != base)
    def _accum():
        out_ref[...] = out_ref[...] + val


def _grouped_ffn(sorted_x, sorted_scores, step_block, step_expert, step_lo,
                 step_hi, W_gate, W_up, W_down):
    n, d_model = sorted_x.shape
    d_ff = W_gate.shape[2]
    num_steps = step_block.shape[0]

    grid_spec = pltpu.PrefetchScalarGridSpec(
        num_scalar_prefetch=4,
        grid=(num_steps,),
        in_specs=[
            pl.BlockSpec((_BM, d_model), lambda t, bl, ex, lo, hi: (bl[t], 0)),
            pl.BlockSpec((1, d_model, d_ff), lambda t, bl, ex, lo, hi: (ex[t], 0, 0)),
            pl.BlockSpec((1, d_model, d_ff), lambda t, bl, ex, lo, hi: (ex[t], 0, 0)),
            pl.BlockSpec((1, d_ff, d_model), lambda t, bl, ex, lo, hi: (ex[t], 0, 0)),
            pl.BlockSpec((_BM, 1), lambda t, bl, ex, lo, hi: (bl[t], 0)),
        ],
        out_specs=pl.BlockSpec((_BM, d_model), lambda t, bl, ex, lo, hi: (bl[t], 0)),
    )
    return pl.pallas_call(
        _ffn_tile_kernel,
        grid_spec=grid_spec,
        out_shape=jax.ShapeDtypeStruct((n, d_model), jnp.float32),
    )(step_block, step_expert, step_lo, step_hi, sorted_x, W_gate, W_up,
      W_down, sorted_scores)


def _sc_combine(cat, pos0, pos1):
    b = pos0.shape[0]
    d = cat.shape[1]
    per_w = b // _SC_WORKERS
    ch = min(64, per_w)
    n_ch = per_w // ch
    groups = d // _LANES
    mesh = plsc.VectorSubcoreMesh(core_axis_name="c", subcore_axis_name="s")

    @functools.partial(
        pl.kernel, mesh=mesh,
        out_type=jax.ShapeDtypeStruct((b, d), jnp.float32),
        scratch_types=[
            pltpu.VMEM((ch,), jnp.int32),
            pltpu.VMEM((ch,), jnp.int32),
            pltpu.VMEM((ch, d), jnp.float32),
            pltpu.VMEM((ch, d), jnp.float32),
            pltpu.SemaphoreType.DMA,
        ],
    )
    def k(cat_hbm, pos0_hbm, pos1_hbm, y_hbm, idx0_v, idx1_v, buf0, buf1, sem):
        wid = lax.axis_index("s") * _SC_CORES + lax.axis_index("c")
        wbase = wid * per_w

        def chunk_body(c, carry):
            base = wbase + c * ch
            pltpu.sync_copy(pos0_hbm.at[pl.ds(base, ch)], idx0_v)
            pltpu.sync_copy(pos1_hbm.at[pl.ds(base, ch)], idx1_v)
            cp0 = pltpu.async_copy(cat_hbm.at[idx0_v], buf0, sem)
            cp1 = pltpu.async_copy(cat_hbm.at[idx1_v], buf1, sem)
            cp0.wait()
            cp1.wait()

            def tok_body(r, carry2):
                for j in range(groups):
                    a = buf0[r, pl.ds(j * _LANES, _LANES)]
                    bb = buf1[r, pl.ds(j * _LANES, _LANES)]
                    buf0[r, pl.ds(j * _LANES, _LANES)] = a + bb
                return carry2

            lax.fori_loop(0, ch, tok_body, 0)
            pltpu.sync_copy(buf0, y_hbm.at[pl.ds(base, ch)])
            return carry

        lax.fori_loop(0, n_ch, chunk_body, 0)

    return k(cat, pos0, pos1)


def kernel(x, topK_indices, topK_scores, W_gate, W_up, W_down):
    batch, k = topK_indices.shape
    num_experts = W_gate.shape[0]
    n = batch * k

    flat_idx = topK_indices.reshape(-1)
    order = jnp.argsort(flat_idx)
    sorted_idx = flat_idx[order]
    sorted_batch = order // k

    counts = jnp.bincount(flat_idx, length=num_experts)
    off = jnp.concatenate([jnp.zeros((1,), jnp.int32),
                           jnp.cumsum(counts).astype(jnp.int32)])

    nb = n // _BM
    e_first = sorted_idx[::_BM]
    e_last = sorted_idx[_BM - 1::_BM]
    nb_tiles = (e_last - e_first + 1).astype(jnp.int32)
    start_t = jnp.concatenate([jnp.zeros((1,), jnp.int32),
                               jnp.cumsum(nb_tiles).astype(jnp.int32)[:-1]])
    total = start_t[-1] + nb_tiles[-1]

    num_steps = nb + num_experts
    ts = jnp.arange(num_steps, dtype=jnp.int32)
    blk = jnp.searchsorted(start_t, ts, side='right').astype(jnp.int32) - 1
    blk = jnp.clip(blk, 0, nb - 1)
    expert = e_first[blk] + (ts - start_t[blk])
    valid = ts < total
    expert = jnp.clip(jnp.where(valid, expert, sorted_idx[-1]), 0, num_experts - 1)
    base = blk * _BM
    lo = jnp.maximum(base, off[expert])
    hi = jnp.minimum(base + _BM, off[expert + 1])
    hi = jnp.maximum(hi, lo)
    lo = jnp.where(valid, lo, base + _BM)
    hi = jnp.where(valid, hi, base + _BM)

    sorted_x = x[sorted_batch]
    sorted_scores = topK_scores.reshape(-1)[order].reshape(n, 1)
    cat = _grouped_ffn(sorted_x, sorted_scores, blk, expert, lo, hi,
                       W_gate, W_up, W_down)

    inv = jnp.zeros((n,), jnp.int32).at[order].set(jnp.arange(n, dtype=jnp.int32))
    pos = inv.reshape(batch, k)
    return _sc_combine(cat, pos[:, 0], pos[:, 1])
